# baseline (speedup 1.0000x reference)
import jax
import jax.numpy as jnp
from jax.experimental import pallas as pl
from jax.experimental.pallas import tpu as pltpu

_N_CHUNKS = 8


def _copy_body(in_ref, out_ref, sem):
    rows = in_ref.shape[0]
    step = rows // _N_CHUNKS
    for i in range(_N_CHUNKS):
        pltpu.make_async_copy(
            in_ref.at[pl.ds(i * step, step)],
            out_ref.at[pl.ds(i * step, step)],
            sem,
        ).start()
    for i in range(_N_CHUNKS):
        pltpu.make_async_copy(
            in_ref.at[pl.ds(i * step, step)],
            out_ref.at[pl.ds(i * step, step)],
            sem,
        ).wait()


def kernel(samples, sample_rate):
    batch, ch, n = samples.shape
    flat = samples.reshape(batch * ch, n)
    out = pl.pallas_call(
        _copy_body,
        out_shape=jax.ShapeDtypeStruct(flat.shape, flat.dtype),
        in_specs=[pl.BlockSpec(memory_space=pltpu.MemorySpace.HBM)],
        out_specs=pl.BlockSpec(memory_space=pltpu.MemorySpace.HBM),
        scratch_shapes=[pltpu.SemaphoreType.DMA],
    )(flat)
    return out.reshape(batch, ch, n)


# speedup vs baseline: 12.3046x; 12.3046x over previous
import jax
import jax.numpy as jnp
from jax.experimental import pallas as pl
from jax.experimental.pallas import tpu as pltpu

_BLOCK_ROWS = 8


def _copy_body(in_ref, out_ref):
    out_ref[...] = in_ref[...]


def kernel(samples, sample_rate):
    batch, ch, n = samples.shape
    flat = samples.reshape(batch * ch, n)
    grid = (flat.shape[0] // _BLOCK_ROWS,)
    out = pl.pallas_call(
        _copy_body,
        out_shape=jax.ShapeDtypeStruct(flat.shape, flat.dtype),
        grid=grid,
        in_specs=[pl.BlockSpec((_BLOCK_ROWS, n), lambda i: (i, 0))],
        out_specs=pl.BlockSpec((_BLOCK_ROWS, n), lambda i: (i, 0)),
    )(flat)
    return out.reshape(batch, ch, n)
